# baseline (speedup 1.0000x reference)
import jax
import jax.numpy as jnp
from jax import lax
from jax.experimental import pallas as pl
from jax.experimental.pallas import tpu as pltpu
from jax.experimental.pallas import tpu_sc as plsc

N = 10000
E = 320000
F = 128
NUM_RELS = 8
NUM_BASES = 4

NPAD = 10240
NC = 2
NS = 16
CH = 128
N0 = 120
N1 = 40
CROWS = 2720
EPADT = CROWS * F
ACC_ROWS = 10112
RPT = ACC_ROWS // NS
RBLK = 512
GBLK = CROWS // (NPAD // RBLK)


def _z_body(coeff_ref, x_ref, w_ref, src_ref, ety_ref, z_ref, g_ref):
    x = x_ref[...]
    ys = [
        jnp.dot(x, w_ref[b], preferred_element_type=jnp.float32)
        for b in range(NUM_BASES)
    ]
    for r in range(NUM_RELS):
        acc = ys[0] * coeff_ref[r, 0]
        for b in range(1, NUM_BASES):
            acc = acc + ys[b] * coeff_ref[r, b]
        z_ref[r] = acc
    g_ref[...] = ety_ref[...] * NPAD + src_ref[...]


def _compute_z(featp, w, coeff, src2d, ety2d):
    grid = NPAD // RBLK
    return pl.pallas_call(
        _z_body,
        grid=(grid,),
        in_specs=[
            pl.BlockSpec(memory_space=pltpu.SMEM),
            pl.BlockSpec((RBLK, F), lambda i: (i, 0)),
            pl.BlockSpec((NUM_BASES, F, F), lambda i: (0, 0, 0)),
            pl.BlockSpec((GBLK, F), lambda i: (i, 0)),
            pl.BlockSpec((GBLK, F), lambda i: (i, 0)),
        ],
        out_specs=[
            pl.BlockSpec((NUM_RELS, RBLK, F), lambda i: (0, i, 0)),
            pl.BlockSpec((GBLK, F), lambda i: (i, 0)),
        ],
        out_shape=[
            jax.ShapeDtypeStruct((NUM_RELS, NPAD, F), jnp.float32),
            jax.ShapeDtypeStruct((CROWS, F), jnp.int32),
        ],
    )(coeff, featp, w, src2d, ety2d)


def _sc_body(z_hbm, g_hbm, dst_hbm, zeros_hbm, out_hbm,
             g_v, dst_v, rows0, acc, sem0):
    cid = lax.axis_index("c")
    sid = lax.axis_index("s")
    nchunk = jnp.where(cid == 0, N0, N1)
    rowbase = jnp.where(cid == 0, sid * N0, NS * N0 + sid * N1)

    pltpu.sync_copy(zeros_hbm.at[pl.ds(sid * RPT, RPT)],
                    acc.at[pl.ds(sid * RPT, RPT)])
    pltpu.sync_copy(g_hbm.at[pl.ds(rowbase, N0)], g_v)
    pltpu.sync_copy(dst_hbm.at[pl.ds(rowbase, N0)], dst_v)
    plsc.subcore_barrier()

    def chunk_body(c, _):
        cp = pltpu.async_copy(z_hbm.at[g_v.at[c]], rows0, sem0)
        cp.wait()
        pltpu.sync_copy(rows0, acc.at[dst_v.at[c]], add=True)
        return 0

    lax.fori_loop(0, nchunk, chunk_body, 0)

    plsc.subcore_barrier()

    pltpu.sync_copy(acc.at[pl.ds(sid * RPT, RPT)],
                    out_hbm.at[cid, pl.ds(sid * RPT, RPT)])


def _sc_aggregate(zflat, g2d, dst2d, zeros):
    mesh = plsc.VectorSubcoreMesh(core_axis_name="c", subcore_axis_name="s")
    run = pl.kernel(
        _sc_body,
        mesh=mesh,
        out_type=jax.ShapeDtypeStruct((NC, ACC_ROWS, F), jnp.float32),
        scratch_types=[
            pltpu.VMEM((N0, CH), jnp.int32),
            pltpu.VMEM((N0, CH), jnp.int32),
            pltpu.VMEM((CH, F), jnp.float32),
            pltpu.VMEM_SHARED((ACC_ROWS, F), jnp.float32),
            pltpu.SemaphoreType.DMA,
        ],
    )
    return run(zflat, g2d, dst2d, zeros)


def _combine_body(a0_ref, a1_ref, x_ref, lw_ref, b_ref, o_ref):
    o_ref[...] = (
        a0_ref[0]
        + a1_ref[0]
        + jnp.dot(x_ref[...], lw_ref[...], preferred_element_type=jnp.float32)
        + b_ref[...]
    )


CBLK = 400


def _combine(accs, feat, loop_weight, h_bias):
    return pl.pallas_call(
        _combine_body,
        grid=(N // CBLK,),
        in_specs=[
            pl.BlockSpec((1, CBLK, F), lambda i: (0, i, 0)),
            pl.BlockSpec((1, CBLK, F), lambda i: (1, i, 0)),
            pl.BlockSpec((CBLK, F), lambda i: (i, 0)),
            pl.BlockSpec((F, F), lambda i: (0, 0)),
            pl.BlockSpec((1, F), lambda i: (0, 0)),
        ],
        out_specs=pl.BlockSpec((CBLK, F), lambda i: (i, 0)),
        out_shape=jax.ShapeDtypeStruct((N, F), jnp.float32),
    )(accs, accs, feat, loop_weight, h_bias.reshape(1, F))


def kernel(feat, edge_index, edge_types, W, coeff, h_bias, loop_weight):
    src = edge_index[0].astype(jnp.int32)
    dst = edge_index[1].astype(jnp.int32)
    ety = edge_types.astype(jnp.int32)

    src_p = jnp.concatenate([src, jnp.zeros((EPADT - E,), jnp.int32)])
    ety_p = jnp.concatenate([ety, jnp.zeros((EPADT - E,), jnp.int32)])
    dst_p = jnp.concatenate(
        [dst, N + (jnp.arange(EPADT - E, dtype=jnp.int32) % (ACC_ROWS - N))])

    src2d = src_p.reshape(CROWS, F)
    ety2d = ety_p.reshape(CROWS, F)
    dst2d = dst_p.reshape(CROWS, F)

    featp = jnp.pad(feat, ((0, NPAD - N), (0, 0)))
    zeros = jnp.zeros((ACC_ROWS, F), jnp.float32)

    z, g2d = _compute_z(featp, W, coeff, src2d, ety2d)
    zflat = z.reshape(NUM_RELS * NPAD, F)
    accs = _sc_aggregate(zflat, g2d, dst2d, zeros)
    return _combine(accs, feat, loop_weight, h_bias)


# speedup vs baseline: 1.3354x; 1.3354x over previous
import jax
import jax.numpy as jnp
from jax import lax
from jax.experimental import pallas as pl
from jax.experimental.pallas import tpu as pltpu
from jax.experimental.pallas import tpu_sc as plsc

N = 10000
E = 320000
F = 128
NUM_RELS = 8
NUM_BASES = 4

NPAD = 10240
NC = 2
NS = 16
NW = NC * NS
CH = 128
NCHUNK = 79
EPT = NCHUNK * CH
EPAD = EPT * NW
ROWS_PER_TILE = NPAD // NS
RBLK = 512


def _z_body(coeff_ref, x_ref, w_ref, z_ref):
    x = x_ref[...]
    ys = [
        jnp.dot(x, w_ref[b], preferred_element_type=jnp.float32)
        for b in range(NUM_BASES)
    ]
    for r in range(NUM_RELS):
        acc = ys[0] * coeff_ref[r, 0]
        for b in range(1, NUM_BASES):
            acc = acc + ys[b] * coeff_ref[r, b]
        z_ref[r] = acc


def _compute_z(featp, w, coeff):
    grid = NPAD // RBLK
    return pl.pallas_call(
        _z_body,
        grid=(grid,),
        in_specs=[
            pl.BlockSpec(memory_space=pltpu.SMEM),
            pl.BlockSpec((RBLK, F), lambda i: (i, 0)),
            pl.BlockSpec((NUM_BASES, F, F), lambda i: (0, 0, 0)),
        ],
        out_specs=pl.BlockSpec((NUM_RELS, RBLK, F), lambda i: (0, i, 0)),
        out_shape=jax.ShapeDtypeStruct((NUM_RELS, NPAD, F), jnp.float32),
    )(coeff, featp, w)


def _sc_body(z_hbm, srcb, etyb, dstb, zeros_hbm, out_hbm,
             src_v, ety_v, dst_v, rows_v, acc, sem):
    cid = lax.axis_index("c")
    sid = lax.axis_index("s")
    wid = sid * NC + cid

    pltpu.sync_copy(zeros_hbm.at[pl.ds(sid * ROWS_PER_TILE, ROWS_PER_TILE)],
                    acc.at[pl.ds(sid * ROWS_PER_TILE, ROWS_PER_TILE)])

    pltpu.sync_copy(srcb.at[wid], src_v)
    pltpu.sync_copy(etyb.at[wid], ety_v)
    pltpu.sync_copy(dstb.at[wid], dst_v)

    def idx_body(i, _):
        off = pl.multiple_of(i * 16, 16)
        s = src_v[pl.ds(off, 16)]
        t = ety_v[pl.ds(off, 16)]
        src_v[pl.ds(off, 16)] = t * NPAD + s
        return 0

    lax.fori_loop(0, EPT // 16, idx_body, 0)

    plsc.subcore_barrier()

    def chunk_body(c, _):
        off = pl.multiple_of(c * CH, CH)
        idx = src_v.at[pl.ds(off, CH)]
        pltpu.async_copy(z_hbm.at[idx], rows_v, sem).wait()
        pltpu.sync_copy(rows_v, acc.at[dst_v.at[c]], add=True)
        return 0

    lax.fori_loop(0, NCHUNK, chunk_body, 0)

    plsc.subcore_barrier()

    pltpu.sync_copy(acc.at[pl.ds(sid * ROWS_PER_TILE, ROWS_PER_TILE)],
                    out_hbm.at[cid, pl.ds(sid * ROWS_PER_TILE, ROWS_PER_TILE)])


def _sc_aggregate(zflat, srcb, etyb, dstb, zeros):
    mesh = plsc.VectorSubcoreMesh(core_axis_name="c", subcore_axis_name="s")
    run = pl.kernel(
        _sc_body,
        mesh=mesh,
        out_type=jax.ShapeDtypeStruct((NC, NPAD, F), jnp.float32),
        scratch_types=[
            pltpu.VMEM((EPT,), jnp.int32),
            pltpu.VMEM((EPT,), jnp.int32),
            pltpu.VMEM((NCHUNK, CH), jnp.int32),
            pltpu.VMEM((CH, F), jnp.float32),
            pltpu.VMEM_SHARED((NPAD, F), jnp.float32),
            pltpu.SemaphoreType.DMA,
        ],
    )
    return run(zflat, srcb, etyb, dstb, zeros)


def _combine_body(a0_ref, a1_ref, x_ref, lw_ref, b_ref, o_ref):
    o_ref[...] = (
        a0_ref[0]
        + a1_ref[0]
        + jnp.dot(x_ref[...], lw_ref[...], preferred_element_type=jnp.float32)
        + b_ref[...]
    )


def _combine(accs, featp, loop_weight, h_bias):
    grid = NPAD // RBLK
    return pl.pallas_call(
        _combine_body,
        grid=(grid,),
        in_specs=[
            pl.BlockSpec((1, RBLK, F), lambda i: (0, i, 0)),
            pl.BlockSpec((1, RBLK, F), lambda i: (1, i, 0)),
            pl.BlockSpec((RBLK, F), lambda i: (i, 0)),
            pl.BlockSpec((F, F), lambda i: (0, 0)),
            pl.BlockSpec((1, F), lambda i: (0, 0)),
        ],
        out_specs=pl.BlockSpec((RBLK, F), lambda i: (i, 0)),
        out_shape=jax.ShapeDtypeStruct((NPAD, F), jnp.float32),
    )(accs, accs, featp, loop_weight, h_bias.reshape(1, F))


def kernel(feat, edge_index, edge_types, W, coeff, h_bias, loop_weight):
    src = edge_index[0].astype(jnp.int32)
    dst = edge_index[1].astype(jnp.int32)
    ety = edge_types.astype(jnp.int32)

    pad = EPAD - E
    src_p = jnp.concatenate([src, jnp.zeros((pad,), jnp.int32)])
    ety_p = jnp.concatenate([ety, jnp.zeros((pad,), jnp.int32)])
    dst_p = jnp.concatenate([dst, jnp.full((pad,), N, jnp.int32)])

    srcb = src_p.reshape(NW, EPT)
    etyb = ety_p.reshape(NW, EPT)
    dstb = dst_p.reshape(NW, NCHUNK, CH)

    featp = jnp.pad(feat, ((0, NPAD - N), (0, 0)))
    zeros = jnp.zeros((NPAD, F), jnp.float32)

    z = _compute_z(featp, W, coeff).reshape(NUM_RELS * NPAD, F)
    accs = _sc_aggregate(z, srcb, etyb, dstb, zeros)
    h = _combine(accs, featp, loop_weight, h_bias)
    return h[:N]


# speedup vs baseline: 1.5801x; 1.1832x over previous
import jax
import jax.numpy as jnp
from jax import lax
from jax.experimental import pallas as pl
from jax.experimental.pallas import tpu as pltpu
from jax.experimental.pallas import tpu_sc as plsc

N = 10000
E = 320000
F = 128
NUM_RELS = 8
NUM_BASES = 4

NPAD = 10240
NC = 2
NS = 16
NW = NC * NS
CH = 128
NCHUNK = 79
EPT = NCHUNK * CH
EPAD = EPT * NW
ROWS_PER_TILE = NPAD // NS
RBLK = 512


def _z_body(coeff_ref, x_ref, w_ref, z_ref, wcat_ref):
    @pl.when(pl.program_id(0) == 0)
    def _():
        for r in range(NUM_RELS):
            acc = w_ref[0] * coeff_ref[r, 0]
            for b in range(1, NUM_BASES):
                acc = acc + w_ref[b] * coeff_ref[r, b]
            wcat_ref[:, r * F:(r + 1) * F] = acc

    y = jnp.dot(x_ref[...], wcat_ref[...], preferred_element_type=jnp.float32)
    for r in range(NUM_RELS):
        z_ref[r] = y[:, r * F:(r + 1) * F]


def _compute_z(featp, w, coeff):
    grid = NPAD // RBLK
    return pl.pallas_call(
        _z_body,
        grid=(grid,),
        in_specs=[
            pl.BlockSpec(memory_space=pltpu.SMEM),
            pl.BlockSpec((RBLK, F), lambda i: (i, 0)),
            pl.BlockSpec((NUM_BASES, F, F), lambda i: (0, 0, 0)),
        ],
        out_specs=pl.BlockSpec((NUM_RELS, RBLK, F), lambda i: (0, i, 0)),
        out_shape=jax.ShapeDtypeStruct((NUM_RELS, NPAD, F), jnp.float32),
        scratch_shapes=[pltpu.VMEM((F, NUM_RELS * F), jnp.float32)],
    )(coeff, featp, w)


def _sc_body(z_hbm, srcb, etyb, dstb, zeros_hbm, out_hbm,
             src_v, ety_v, dst_v, rows_v, acc, sem):
    cid = lax.axis_index("c")
    sid = lax.axis_index("s")
    wid = sid * NC + cid

    pltpu.sync_copy(zeros_hbm.at[pl.ds(sid * ROWS_PER_TILE, ROWS_PER_TILE)],
                    acc.at[pl.ds(sid * ROWS_PER_TILE, ROWS_PER_TILE)])

    pltpu.sync_copy(srcb.at[wid], src_v)
    pltpu.sync_copy(etyb.at[wid], ety_v)
    pltpu.sync_copy(dstb.at[wid], dst_v)

    def idx_body(i, _):
        off = pl.multiple_of(i * 16, 16)
        s = src_v[pl.ds(off, 16)]
        t = ety_v[pl.ds(off, 16)]
        src_v[pl.ds(off, 16)] = t * NPAD + s
        return 0

    lax.fori_loop(0, EPT // 16, idx_body, 0)

    plsc.subcore_barrier()

    def chunk_body(c, _):
        off = pl.multiple_of(c * CH, CH)
        idx = src_v.at[pl.ds(off, CH)]
        pltpu.async_copy(z_hbm.at[idx], rows_v, sem).wait()
        pltpu.sync_copy(rows_v, acc.at[dst_v.at[c]], add=True)
        return 0

    lax.fori_loop(0, NCHUNK, chunk_body, 0)

    plsc.subcore_barrier()

    pltpu.sync_copy(acc.at[pl.ds(sid * ROWS_PER_TILE, ROWS_PER_TILE)],
                    out_hbm.at[cid, pl.ds(sid * ROWS_PER_TILE, ROWS_PER_TILE)])


def _sc_aggregate(zflat, srcb, etyb, dstb, zeros):
    mesh = plsc.VectorSubcoreMesh(core_axis_name="c", subcore_axis_name="s")
    run = pl.kernel(
        _sc_body,
        mesh=mesh,
        out_type=jax.ShapeDtypeStruct((NC, NPAD, F), jnp.float32),
        scratch_types=[
            pltpu.VMEM((EPT,), jnp.int32),
            pltpu.VMEM((EPT,), jnp.int32),
            pltpu.VMEM((NCHUNK, CH), jnp.int32),
            pltpu.VMEM((CH, F), jnp.float32),
            pltpu.VMEM_SHARED((NPAD, F), jnp.float32),
            pltpu.SemaphoreType.DMA,
        ],
    )
    return run(zflat, srcb, etyb, dstb, zeros)


def _combine_body(a0_ref, a1_ref, x_ref, lw_ref, b_ref, o_ref):
    o_ref[...] = (
        a0_ref[0]
        + a1_ref[0]
        + jnp.dot(x_ref[...], lw_ref[...], preferred_element_type=jnp.float32)
        + b_ref[...]
    )


CBLK = 400


def _combine(accs, feat, loop_weight, h_bias):
    return pl.pallas_call(
        _combine_body,
        grid=(N // CBLK,),
        in_specs=[
            pl.BlockSpec((1, CBLK, F), lambda i: (0, i, 0)),
            pl.BlockSpec((1, CBLK, F), lambda i: (1, i, 0)),
            pl.BlockSpec((CBLK, F), lambda i: (i, 0)),
            pl.BlockSpec((F, F), lambda i: (0, 0)),
            pl.BlockSpec((1, F), lambda i: (0, 0)),
        ],
        out_specs=pl.BlockSpec((CBLK, F), lambda i: (i, 0)),
        out_shape=jax.ShapeDtypeStruct((N, F), jnp.float32),
    )(accs, accs, feat, loop_weight, h_bias.reshape(1, F))


def kernel(feat, edge_index, edge_types, W, coeff, h_bias, loop_weight):
    src = edge_index[0].astype(jnp.int32)
    dst = edge_index[1].astype(jnp.int32)
    ety = edge_types.astype(jnp.int32)

    pad = EPAD - E
    src_p = jnp.concatenate([src, jnp.zeros((pad,), jnp.int32)])
    ety_p = jnp.concatenate([ety, jnp.zeros((pad,), jnp.int32)])
    dst_p = jnp.concatenate([dst, jnp.full((pad,), N, jnp.int32)])

    srcb = src_p.reshape(NW, EPT)
    etyb = ety_p.reshape(NW, EPT)
    dstb = dst_p.reshape(NW, NCHUNK, CH)

    featp = jnp.pad(feat, ((0, NPAD - N), (0, 0)))
    zeros = jnp.zeros((NPAD, F), jnp.float32)

    z = _compute_z(featp, W, coeff).reshape(NUM_RELS * NPAD, F)
    accs = _sc_aggregate(z, srcb, etyb, dstb, zeros)
    return _combine(accs, feat, loop_weight, h_bias)
